# Initial kernel scaffold; baseline (speedup 1.0000x reference)
import jax
import jax.numpy as jnp
from jax.experimental import pallas as pl


def kernel(input_ids, embed_table, W, b):
    raise NotImplementedError("write your pallas kernel here")


# speedup vs baseline: 1.9260x; 1.9260x over previous
import functools

import jax
import jax.numpy as jnp
from jax import lax
from jax.experimental import pallas as pl
from jax.experimental.pallas import tpu as pltpu
from jax.experimental.pallas import tpu_sc as plsc

VOCAB = 1000
HIDDEN = 256

NC = 2
NS = 16
NW = NC * NS

C = 128
NCH = 50
D = HIDDEN


def _proj_kernel(e_ref, w_ref, b_ref, out_ref):
    out_ref[...] = lax.dot_general(
        e_ref[...], w_ref[...],
        dimension_numbers=(((1,), (1,)), ((), ())),
        preferred_element_type=jnp.float32,
    ) + b_ref[...]


def _project_table(E, W, b2d):
    return pl.pallas_call(
        _proj_kernel,
        out_shape=jax.ShapeDtypeStruct((VOCAB, HIDDEN), jnp.float32),
    )(E, W, b2d)


def _gather_body(table_hbm, idx_hbm, out_hbm, idx_v, rows_v, sem):
    wid = lax.axis_index("s") * NC + lax.axis_index("c")
    pltpu.sync_copy(idx_hbm.at[wid], idx_v)

    def body(ch, carry):
        pltpu.async_copy(table_hbm.at[idx_v.at[ch]], rows_v, sem).wait()
        pltpu.sync_copy(rows_v, out_hbm.at[wid, ch])
        return carry

    lax.fori_loop(0, NCH, body, 0)


_gather = functools.partial(
    pl.kernel,
    out_type=jax.ShapeDtypeStruct((NW, NCH, C, D), jnp.float32),
    mesh=plsc.VectorSubcoreMesh(
        core_axis_name="c", subcore_axis_name="s",
        num_cores=NC, num_subcores=NS),
    scratch_types=[
        pltpu.VMEM((NCH, C), jnp.int32),
        pltpu.VMEM((C, D), jnp.float32),
        pltpu.SemaphoreType.DMA,
    ],
)(_gather_body)


def kernel(input_ids, embed_table, W, b):
    P = _project_table(embed_table, W, b.reshape(1, HIDDEN))
    idx = input_ids.reshape(NW, NCH, C).astype(jnp.int32)
    out = _gather(P, idx)
    return out.reshape(input_ids.shape[0], input_ids.shape[1], HIDDEN)


# speedup vs baseline: 1.9679x; 1.0218x over previous
import functools

import jax
import jax.numpy as jnp
from jax import lax
from jax.experimental import pallas as pl
from jax.experimental.pallas import tpu as pltpu
from jax.experimental.pallas import tpu_sc as plsc

VOCAB = 1000
HIDDEN = 256

NC = 2
NS = 16
NW = NC * NS

C = 128
NCH = 50
D = HIDDEN


def _proj_kernel(e_ref, w_ref, b_ref, out_ref):
    out_ref[...] = lax.dot_general(
        e_ref[...], w_ref[...],
        dimension_numbers=(((1,), (1,)), ((), ())),
        preferred_element_type=jnp.float32,
    ) + b_ref[...]


def _project_table(E, W, b2d):
    return pl.pallas_call(
        _proj_kernel,
        out_shape=jax.ShapeDtypeStruct((VOCAB, HIDDEN), jnp.float32),
    )(E, W, b2d)


def _gather_body(table_hbm, idx_hbm, out_hbm, idx_v,
                 rows0, rows1, gsem0, gsem1, ssem0, ssem1):
    wid = lax.axis_index("s") * NC + lax.axis_index("c")
    rows = (rows0, rows1)
    gsem = (gsem0, gsem1)
    ssem = (ssem0, ssem1)
    pltpu.sync_copy(idx_hbm.at[wid], idx_v)

    g0 = pltpu.async_copy(table_hbm.at[idx_v.at[0]], rows0, gsem0)
    g1 = pltpu.async_copy(table_hbm.at[idx_v.at[1]], rows1, gsem1)
    del g0, g1

    @pl.loop(0, NCH, step=2)
    def _(ch):
        for b in range(2):
            cur = ch + b
            pltpu.make_async_copy(table_hbm.at[idx_v.at[cur]],
                                  rows[b], gsem[b]).wait()
            pltpu.async_copy(rows[b], out_hbm.at[wid, cur], ssem[b])
            bp = b ^ 1
            prev = cur - 1

            @pl.when(prev >= 0)
            def _():
                pltpu.make_async_copy(rows[bp], out_hbm.at[wid, prev],
                                      ssem[bp]).wait()

            @pl.when(jnp.logical_and(cur >= 1, cur + 1 < NCH))
            def _():
                pltpu.async_copy(table_hbm.at[idx_v.at[cur + 1]],
                                 rows[bp], gsem[bp])

    pltpu.make_async_copy(rows[1], out_hbm.at[wid, NCH - 1], ssem[1]).wait()


@functools.cache
def _gather():
    return pl.kernel(
        _gather_body,
        out_type=jax.ShapeDtypeStruct((NW, NCH, C, D), jnp.float32),
        mesh=plsc.VectorSubcoreMesh(
            core_axis_name="c", subcore_axis_name="s",
            num_cores=NC, num_subcores=NS),
        scratch_types=[
            pltpu.VMEM((NCH, C), jnp.int32),
            pltpu.VMEM((C, D), jnp.float32),
            pltpu.VMEM((C, D), jnp.float32),
            pltpu.SemaphoreType.DMA,
            pltpu.SemaphoreType.DMA,
            pltpu.SemaphoreType.DMA,
            pltpu.SemaphoreType.DMA,
        ],
    )


def kernel(input_ids, embed_table, W, b):
    P = _project_table(embed_table, W, b.reshape(1, HIDDEN))
    idx = input_ids.reshape(NW, NCH, C).astype(jnp.int32)
    out = _gather()(P, idx)
    return out.reshape(input_ids.shape[0], input_ids.shape[1], HIDDEN)
